# baseline (speedup 1.0000x reference)
import jax
import jax.numpy as jnp
from jax import lax
from jax.experimental import pallas as pl
from jax.experimental.pallas import tpu as pltpu
from jax.experimental.pallas import tpu_sc as plsc

_NUM_TOKENS = 512
_DIM = 1024
_NUM_CHAPTERS = 16
_T = 32
_BATCH = 2048
_K = 2
_NPAIRS = _BATCH * _K
_NC = 2
_NS = 16
_HALF = _NPAIRS // _NC
_PPT = _HALF // _NS
_S_SPLIT = 52
_WINDOW = 8


def _sc_gather_kernel(mem_hbm, cidx_hbm, out_hbm, aidx_hbm,
                      bank, mychap, cidx_v, aidx_v,
                      out_sem, bank_out_sem, in_sem, chap_sem):
    cid = lax.axis_index("c")
    sid = lax.axis_index("s")
    half = cid * _HALF

    pltpu.async_copy(cidx_hbm.at[pl.ds(half, _HALF)], cidx_v, in_sem)
    pltpu.async_copy(mem_hbm.at[pl.ds(sid * _T, _T)], mychap, chap_sem)

    @pl.when(sid == 0)
    def _():
        pltpu.sync_copy(mem_hbm, bank)

    plsc.subcore_barrier()
    pltpu.make_async_copy(cidx_hbm.at[pl.ds(half, _HALF)], cidx_v,
                          in_sem).wait()

    iota = lax.broadcasted_iota(jnp.int32, (16,), 0)
    descs = []
    for g in range(_PPT // 16):
        cvec = cidx_v[pl.ds(sid * _PPT + g * 16, 16)]
        for l in range(16):
            p = g * 16 + l
            row0 = cvec[l] * _T
            lo = row0 + iota
            aidx_v[p, pl.ds(0, 16)] = lo
            aidx_v[p, pl.ds(16, 16)] = lo + 16
            if p < _S_SPLIT:
                d = pltpu.async_copy(bank.at[pl.ds(row0, _T)],
                                     out_hbm.at[half + sid * _PPT + p],
                                     bank_out_sem)
                descs.append(d)
                if len(descs) > _WINDOW:
                    descs.pop(0).wait()
    pltpu.sync_copy(aidx_v, aidx_hbm.at[pl.ds(half + sid * _PPT, _PPT)])

    pltpu.make_async_copy(mem_hbm.at[pl.ds(sid * _T, _T)], mychap,
                          chap_sem).wait()

    def scan_body(g, cnt):
        vec = cidx_v[pl.ds(g * 16, 16)]
        gm = (g % (_PPT // 16)) * 16
        for l in range(16):
            c = vec[l]
            hit = (c == sid) & (gm + l >= _S_SPLIT)

            @pl.when(hit)
            def _():
                pltpu.async_copy(mychap, out_hbm.at[half + g * 16 + l],
                                 out_sem)

            cnt = jnp.where(hit, cnt + 1, cnt)
        return cnt

    n_served = lax.fori_loop(0, _HALF // 16, scan_body, jnp.int32(0))

    for d in descs:
        d.wait()

    def drain_body(i, carry):
        pltpu.make_async_copy(mem_hbm.at[pl.ds(0, _T)], mychap,
                              out_sem).wait()
        return carry

    lax.fori_loop(0, n_served, drain_body, jnp.int32(0))


def kernel(memory, chapter_indices):
    cidx_flat = chapter_indices.reshape(_NPAIRS).astype(jnp.int32)
    mesh = plsc.VectorSubcoreMesh(core_axis_name="c", subcore_axis_name="s")
    gathered, aidx = pl.kernel(
        _sc_gather_kernel,
        out_type=(
            jax.ShapeDtypeStruct((_NPAIRS, _T, _DIM), jnp.float32),
            jax.ShapeDtypeStruct((_NPAIRS, _T), jnp.int32),
        ),
        mesh=mesh,
        scratch_types=[
            pltpu.VMEM_SHARED((_NUM_TOKENS, _DIM), jnp.float32),
            pltpu.VMEM((_T, _DIM), jnp.float32),
            pltpu.VMEM((_HALF,), jnp.int32),
            pltpu.VMEM((_PPT, _T), jnp.int32),
            pltpu.SemaphoreType.DMA,
            pltpu.SemaphoreType.DMA,
            pltpu.SemaphoreType.DMA,
            pltpu.SemaphoreType.DMA,
        ],
    )(memory, cidx_flat)
    return (gathered.reshape(_BATCH, _K * _T, _DIM),
            aidx.reshape(_BATCH, _K * _T).astype(chapter_indices.dtype))


# speedup vs baseline: 1.1822x; 1.1822x over previous
import jax
import jax.numpy as jnp
from jax import lax
from jax.experimental import pallas as pl
from jax.experimental.pallas import tpu as pltpu
from jax.experimental.pallas import tpu_sc as plsc

_NUM_TOKENS = 512
_DIM = 1024
_NUM_CHAPTERS = 16
_T = 32
_BATCH = 2048
_K = 2
_NPAIRS = _BATCH * _K
_NC = 2
_NS = 16
_HALF = _NPAIRS // _NC
_PPT = _HALF // _NS
_S_SPLIT = 52
_WINDOW = 8


def _sc_gather_kernel(mem_hbm, cidx_hbm, out_hbm, aidx_hbm,
                      bank, mychap, cidx_v, aidx_v,
                      out_sem, bank_out_sem, in_sem, chap_sem):
    cid = lax.axis_index("c")
    sid = lax.axis_index("s")
    half = cid * _HALF

    pltpu.async_copy(cidx_hbm.at[pl.ds(half, _HALF)], cidx_v, in_sem)
    pltpu.async_copy(mem_hbm.at[pl.ds(sid * _T, _T)], mychap, chap_sem)

    @pl.when(sid == 0)
    def _():
        pltpu.sync_copy(mem_hbm, bank)

    plsc.subcore_barrier()
    pltpu.make_async_copy(cidx_hbm.at[pl.ds(half, _HALF)], cidx_v,
                          in_sem).wait()

    iota = lax.broadcasted_iota(jnp.int32, (16,), 0)
    for g in range(_PPT // 16):
        cvec = cidx_v[pl.ds(sid * _PPT + g * 16, 16)]
        for l in range(16):
            p = g * 16 + l
            row0 = cvec[l] * _T
            lo = row0 + iota
            aidx_v[p, pl.ds(0, 16)] = lo
            aidx_v[p, pl.ds(16, 16)] = lo + 16
    pltpu.sync_copy(aidx_v, aidx_hbm.at[pl.ds(half + sid * _PPT, _PPT)])

    pltpu.make_async_copy(mem_hbm.at[pl.ds(sid * _T, _T)], mychap,
                          chap_sem).wait()

    def scan_body(g, cnt):
        vec = cidx_v[pl.ds(g * 16, 16)]
        gm = (g % (_PPT // 16)) * 16
        for l in range(16):
            c = vec[l]
            hit = (c == sid) & (gm + l >= _S_SPLIT)

            @pl.when(hit)
            def _():
                pltpu.async_copy(mychap, out_hbm.at[half + g * 16 + l],
                                 out_sem)

            cnt = jnp.where(hit, cnt + 1, cnt)
        return cnt

    n_served = lax.fori_loop(0, _HALF // 16, scan_body, jnp.int32(0))

    descs = []
    for g in range(_PPT // 16):
        cvec = cidx_v[pl.ds(sid * _PPT + g * 16, 16)]
        for l in range(16):
            p = g * 16 + l
            if p < _S_SPLIT:
                d = pltpu.async_copy(bank.at[pl.ds(cvec[l] * _T, _T)],
                                     out_hbm.at[half + sid * _PPT + p],
                                     bank_out_sem)
                descs.append(d)
                if len(descs) > _WINDOW:
                    descs.pop(0).wait()
    for d in descs:
        d.wait()

    def drain_body(i, carry):
        pltpu.make_async_copy(mem_hbm.at[pl.ds(0, _T)], mychap,
                              out_sem).wait()
        return carry

    lax.fori_loop(0, n_served, drain_body, jnp.int32(0))


def kernel(memory, chapter_indices):
    cidx_flat = chapter_indices.reshape(_NPAIRS).astype(jnp.int32)
    mesh = plsc.VectorSubcoreMesh(core_axis_name="c", subcore_axis_name="s")
    gathered, aidx = pl.kernel(
        _sc_gather_kernel,
        out_type=(
            jax.ShapeDtypeStruct((_NPAIRS, _T, _DIM), jnp.float32),
            jax.ShapeDtypeStruct((_NPAIRS, _T), jnp.int32),
        ),
        mesh=mesh,
        scratch_types=[
            pltpu.VMEM_SHARED((_NUM_TOKENS, _DIM), jnp.float32),
            pltpu.VMEM((_T, _DIM), jnp.float32),
            pltpu.VMEM((_HALF,), jnp.int32),
            pltpu.VMEM((_PPT, _T), jnp.int32),
            pltpu.SemaphoreType.DMA,
            pltpu.SemaphoreType.DMA,
            pltpu.SemaphoreType.DMA,
            pltpu.SemaphoreType.DMA,
        ],
    )(memory, cidx_flat)
    return (gathered.reshape(_BATCH, _K * _T, _DIM),
            aidx.reshape(_BATCH, _K * _T).astype(chapter_indices.dtype))


# speedup vs baseline: 1.1828x; 1.0005x over previous
import jax
import jax.numpy as jnp
from jax import lax
from jax.experimental import pallas as pl
from jax.experimental.pallas import tpu as pltpu
from jax.experimental.pallas import tpu_sc as plsc
from jax._src.pallas import mpmd

_NUM_TOKENS = 512
_DIM = 1024
_NUM_CHAPTERS = 16
_T = 32
_BATCH = 2048
_K = 2
_NPAIRS = _BATCH * _K
_NC = 2
_NS = 16
_HALF = _NPAIRS // _NC
_PPT = _HALF // _NS
_S_SPLIT = 51
_WINDOW = 8


def _tec_fn(mem_hbm, cidx_hbm, out_hbm, aidx_hbm,
            bank, mychap, cidx_v, aidx_v, ids_smem,
            out_sem, in_sem, chap_sem, scs_out_sem):
    cid = lax.axis_index("c")
    sid = lax.axis_index("s")
    half = cid * _HALF

    pltpu.async_copy(cidx_hbm.at[pl.ds(half, _HALF)], cidx_v, in_sem)
    pltpu.async_copy(mem_hbm.at[pl.ds(sid * _T, _T)], mychap, chap_sem)
    pltpu.make_async_copy(cidx_hbm.at[pl.ds(half, _HALF)], cidx_v,
                          in_sem).wait()

    iota = lax.broadcasted_iota(jnp.int32, (16,), 0)
    for g in range(_PPT // 16):
        cvec = cidx_v[pl.ds(sid * _PPT + g * 16, 16)]
        for l in range(16):
            p = g * 16 + l
            row0 = cvec[l] * _T
            lo = row0 + iota
            aidx_v[p, pl.ds(0, 16)] = lo
            aidx_v[p, pl.ds(16, 16)] = lo + 16
    pltpu.sync_copy(aidx_v, aidx_hbm.at[pl.ds(half + sid * _PPT, _PPT)])

    pltpu.make_async_copy(mem_hbm.at[pl.ds(sid * _T, _T)], mychap,
                          chap_sem).wait()

    def scan_body(g, cnt):
        vec = cidx_v[pl.ds(g * 16, 16)]
        gm = (g % (_PPT // 16)) * 16
        for l in range(16):
            c = vec[l]
            hit = (c == sid) & (gm + l >= _S_SPLIT)

            @pl.when(hit)
            def _():
                pltpu.async_copy(mychap, out_hbm.at[half + g * 16 + l],
                                 out_sem)

            cnt = jnp.where(hit, cnt + 1, cnt)
        return cnt

    n_served = lax.fori_loop(0, _HALF // 16, scan_body, jnp.int32(0))

    def drain_body(i, carry):
        pltpu.make_async_copy(mem_hbm.at[pl.ds(0, _T)], mychap,
                              out_sem).wait()
        return carry

    lax.fori_loop(0, n_served, drain_body, jnp.int32(0))


def _scs_fn(mem_hbm, cidx_hbm, out_hbm, aidx_hbm,
            bank, mychap, cidx_v, aidx_v, ids_smem,
            out_sem, in_sem, chap_sem, scs_out_sem):
    cid = lax.axis_index("c")
    half = cid * _HALF

    pltpu.sync_copy(mem_hbm, bank)

    descs = []
    for t in range(_NS):
        pltpu.sync_copy(cidx_hbm.at[pl.ds(half + t * _PPT, _PPT)],
                        ids_smem)
        for p in range(_S_SPLIT):
            row0 = ids_smem[p] * _T
            d = pltpu.async_copy(bank.at[pl.ds(row0, _T)],
                                 out_hbm.at[half + t * _PPT + p],
                                 scs_out_sem)
            descs.append(d)
            if len(descs) > _WINDOW:
                descs.pop(0).wait()
    for d in descs:
        d.wait()


def kernel(memory, chapter_indices):
    cidx_flat = chapter_indices.reshape(_NPAIRS).astype(jnp.int32)
    vec_mesh = plsc.VectorSubcoreMesh(core_axis_name="c",
                                      subcore_axis_name="s")
    scs_mesh = plsc.ScalarSubcoreMesh(axis_name="c", num_cores=_NC)
    gathered, aidx = mpmd.mpmd_map(
        [(scs_mesh, _scs_fn), (vec_mesh, _tec_fn)],
        out_types=(
            jax.ShapeDtypeStruct((_NPAIRS, _T, _DIM), jnp.float32),
            jax.ShapeDtypeStruct((_NPAIRS, _T), jnp.int32),
        ),
        scratch_types=[
            pltpu.VMEM_SHARED((_NUM_TOKENS, _DIM), jnp.float32),
            (pltpu.VMEM @ vec_mesh)((_T, _DIM), jnp.float32),
            (pltpu.VMEM @ vec_mesh)((_HALF,), jnp.int32),
            (pltpu.VMEM @ vec_mesh)((_PPT, _T), jnp.int32),
            (pltpu.SMEM @ scs_mesh)((_PPT,), jnp.int32),
            pltpu.SemaphoreType.DMA @ vec_mesh,
            pltpu.SemaphoreType.DMA @ vec_mesh,
            pltpu.SemaphoreType.DMA @ vec_mesh,
            pltpu.SemaphoreType.DMA @ scs_mesh,
        ],
    )(memory, cidx_flat)
    return (gathered.reshape(_BATCH, _K * _T, _DIM),
            aidx.reshape(_BATCH, _K * _T).astype(chapter_indices.dtype))


# speedup vs baseline: 1.1977x; 1.0126x over previous
import jax
import jax.numpy as jnp
from jax import lax
from jax.experimental import pallas as pl
from jax.experimental.pallas import tpu as pltpu
from jax.experimental.pallas import tpu_sc as plsc

_NUM_TOKENS = 512
_DIM = 1024
_NUM_CHAPTERS = 16
_T = 32
_BATCH = 2048
_K = 2
_NPAIRS = _BATCH * _K
_NC = 2
_NS = 16
_HALF = _NPAIRS // _NC
_PPT = _HALF // _NS


def _sc_gather_kernel(mem_hbm, cidx_hbm, out_hbm, aidx_hbm,
                      mychap, cidx_v, aidx_v, out_sem, in_sem, chap_sem):
    cid = lax.axis_index("c")
    sid = lax.axis_index("s")
    half = cid * _HALF

    pltpu.async_copy(cidx_hbm.at[pl.ds(half, _HALF)], cidx_v, in_sem)
    pltpu.async_copy(mem_hbm.at[pl.ds(sid * _T, _T)], mychap, chap_sem)
    pltpu.make_async_copy(cidx_hbm.at[pl.ds(half, _HALF)], cidx_v,
                          in_sem).wait()
    pltpu.make_async_copy(mem_hbm.at[pl.ds(sid * _T, _T)], mychap,
                          chap_sem).wait()

    def scan_body(g, cnt):
        vec = cidx_v[pl.ds(g * 16, 16)]
        for l in range(16):
            c = vec[l]
            hit = c == sid

            @pl.when(hit)
            def _():
                pltpu.async_copy(mychap, out_hbm.at[half + g * 16 + l],
                                 out_sem)

            cnt = jnp.where(hit, cnt + 1, cnt)
        return cnt

    n_served = lax.fori_loop(0, _HALF // 16, scan_body, jnp.int32(0))

    iota = lax.broadcasted_iota(jnp.int32, (16,), 0)
    for g in range(_PPT // 16):
        cvec = cidx_v[pl.ds(sid * _PPT + g * 16, 16)]
        for l in range(16):
            p = g * 16 + l
            row0 = cvec[l] * _T
            lo = row0 + iota
            aidx_v[p, pl.ds(0, 16)] = lo
            aidx_v[p, pl.ds(16, 16)] = lo + 16
    pltpu.sync_copy(aidx_v, aidx_hbm.at[pl.ds(half + sid * _PPT, _PPT)])

    def drain_body(i, carry):
        pltpu.make_async_copy(mem_hbm.at[pl.ds(0, _T)], mychap,
                              out_sem).wait()
        return carry

    lax.fori_loop(0, n_served, drain_body, jnp.int32(0))


def kernel(memory, chapter_indices):
    cidx_flat = chapter_indices.reshape(_NPAIRS).astype(jnp.int32)
    mesh = plsc.VectorSubcoreMesh(core_axis_name="c", subcore_axis_name="s")
    gathered, aidx = pl.kernel(
        _sc_gather_kernel,
        out_type=(
            jax.ShapeDtypeStruct((_NPAIRS, _T, _DIM), jnp.float32),
            jax.ShapeDtypeStruct((_NPAIRS, _T), jnp.int32),
        ),
        mesh=mesh,
        scratch_types=[
            pltpu.VMEM((_T, _DIM), jnp.float32),
            pltpu.VMEM((_HALF,), jnp.int32),
            pltpu.VMEM((_PPT, _T), jnp.int32),
            pltpu.SemaphoreType.DMA,
            pltpu.SemaphoreType.DMA,
            pltpu.SemaphoreType.DMA,
        ],
    )(memory, cidx_flat)
    return (gathered.reshape(_BATCH, _K * _T, _DIM),
            aidx.reshape(_BATCH, _K * _T).astype(chapter_indices.dtype))


# speedup vs baseline: 1.1996x; 1.0015x over previous
import jax
import jax.numpy as jnp
from jax import lax
from jax.experimental import pallas as pl
from jax.experimental.pallas import tpu as pltpu
from jax.experimental.pallas import tpu_sc as plsc

_NUM_TOKENS = 512
_DIM = 1024
_NUM_CHAPTERS = 16
_T = 32
_BATCH = 2048
_K = 2
_NPAIRS = _BATCH * _K
_NC = 2
_NS = 16
_HALF = _NPAIRS // _NC
_PPT = _HALF // _NS


def _sc_gather_kernel(mem_hbm, cidx_hbm, out_hbm, aidx_hbm,
                      mychap, cidx_v, aidx_v, out_sem, in_sem, chap_sem):
    cid = lax.axis_index("c")
    sid = lax.axis_index("s")
    half = cid * _HALF

    pltpu.async_copy(cidx_hbm.at[pl.ds(half, _HALF)], cidx_v, in_sem)
    pltpu.async_copy(mem_hbm.at[pl.ds(sid * _T, _T)], mychap, chap_sem)
    pltpu.make_async_copy(cidx_hbm.at[pl.ds(half, _HALF)], cidx_v,
                          in_sem).wait()

    iota = lax.broadcasted_iota(jnp.int32, (16,), 0)
    for g in range(_PPT // 16):
        cvec = cidx_v[pl.ds(sid * _PPT + g * 16, 16)]
        for l in range(16):
            p = g * 16 + l
            row0 = cvec[l] * _T
            lo = row0 + iota
            aidx_v[p, pl.ds(0, 16)] = lo
            aidx_v[p, pl.ds(16, 16)] = lo + 16
    pltpu.sync_copy(aidx_v, aidx_hbm.at[pl.ds(half + sid * _PPT, _PPT)])

    pltpu.make_async_copy(mem_hbm.at[pl.ds(sid * _T, _T)], mychap,
                          chap_sem).wait()

    def scan_body(g, cnt):
        vec = cidx_v[pl.ds(g * 16, 16)]
        for l in range(16):
            c = vec[l]
            hit = c == sid

            @pl.when(hit)
            def _():
                pltpu.async_copy(mychap, out_hbm.at[half + g * 16 + l],
                                 out_sem)

            cnt = jnp.where(hit, cnt + 1, cnt)
        return cnt

    n_served = lax.fori_loop(0, _HALF // 16, scan_body, jnp.int32(0))

    def drain_body(i, carry):
        pltpu.make_async_copy(mem_hbm.at[pl.ds(0, _T)], mychap,
                              out_sem).wait()
        return carry

    lax.fori_loop(0, n_served, drain_body, jnp.int32(0))


def kernel(memory, chapter_indices):
    cidx_flat = chapter_indices.reshape(_NPAIRS).astype(jnp.int32)
    mesh = plsc.VectorSubcoreMesh(core_axis_name="c", subcore_axis_name="s")
    gathered, aidx = pl.kernel(
        _sc_gather_kernel,
        out_type=(
            jax.ShapeDtypeStruct((_NPAIRS, _T, _DIM), jnp.float32),
            jax.ShapeDtypeStruct((_NPAIRS, _T), jnp.int32),
        ),
        mesh=mesh,
        scratch_types=[
            pltpu.VMEM((_T, _DIM), jnp.float32),
            pltpu.VMEM((_HALF,), jnp.int32),
            pltpu.VMEM((_PPT, _T), jnp.int32),
            pltpu.SemaphoreType.DMA,
            pltpu.SemaphoreType.DMA,
            pltpu.SemaphoreType.DMA,
        ],
    )(memory, cidx_flat)
    return (gathered.reshape(_BATCH, _K * _T, _DIM),
            aidx.reshape(_BATCH, _K * _T).astype(chapter_indices.dtype))
